# baseline (speedup 1.0000x reference)
import jax
import jax.numpy as jnp
from jax.experimental import pallas as pl


def _sum_block(x_ref, o_ref):
    o_ref[...] = jnp.sum(x_ref[...], axis=-1, keepdims=True)


def kernel(logits):
    b, v = logits.shape
    va = 99968
    xa = logits[:, :va]
    br = 16
    s = pl.pallas_call(
        _sum_block,
        grid=(b // br,),
        in_specs=[pl.BlockSpec((br, va), lambda i: (i, 0))],
        out_specs=pl.BlockSpec((br, 1), lambda i: (i, 0)),
        out_shape=jax.ShapeDtypeStruct((b, 1), logits.dtype),
    )(xa)
    return logits - 1.0 + s * 0.0


# speedup vs baseline: 1.0073x; 1.0073x over previous
import jax
import jax.numpy as jnp
from jax.experimental import pallas as pl
from jax.experimental.pallas import tpu as pltpu

RPC = 8
K = 4


def _body(x_hbm, o_hbm, in_buf, out_buf, in_sems, out_sems):
    i = pl.program_id(0)
    nchunk = pl.num_programs(0)
    s = jax.lax.rem(i, K)

    @pl.when(i == 0)
    def _prologue():
        for k in range(K):
            pltpu.make_async_copy(
                x_hbm.at[pl.ds(k * RPC, RPC)], in_buf.at[k], in_sems.at[k]
            ).start()

    pltpu.make_async_copy(
        x_hbm.at[pl.ds(i * RPC, RPC)], in_buf.at[s], in_sems.at[s]
    ).wait()

    x = in_buf[s]
    m = jnp.max(x, axis=-1, keepdims=True)
    ssum = jnp.sum(jnp.exp(x - m), axis=-1, keepdims=True)
    lse = m + jnp.log(ssum)

    @pl.when(i >= K)
    def _drain_prev():
        pltpu.make_async_copy(
            out_buf.at[s], o_hbm.at[pl.ds((i - K) * RPC, RPC)], out_sems.at[s]
        ).wait()

    out_buf[s] = x - lse
    pltpu.make_async_copy(
        out_buf.at[s], o_hbm.at[pl.ds(i * RPC, RPC)], out_sems.at[s]
    ).start()

    @pl.when(i + K < nchunk)
    def _refill():
        pltpu.make_async_copy(
            x_hbm.at[pl.ds((i + K) * RPC, RPC)], in_buf.at[s], in_sems.at[s]
        ).start()

    @pl.when(i == nchunk - 1)
    def _epilogue():
        for k in range(K):
            j = nchunk - K + k
            sk = jax.lax.rem(j, K)
            pltpu.make_async_copy(
                out_buf.at[sk], o_hbm.at[pl.ds(j * RPC, RPC)], out_sems.at[sk]
            ).wait()


def kernel(logits):
    b, v = logits.shape
    nchunk = b // RPC
    return pl.pallas_call(
        _body,
        grid=(nchunk,),
        in_specs=[pl.BlockSpec(memory_space=pltpu.HBM)],
        out_specs=pl.BlockSpec(memory_space=pltpu.HBM),
        out_shape=jax.ShapeDtypeStruct((b, v), logits.dtype),
        scratch_shapes=[
            pltpu.VMEM((K, RPC, v), jnp.float32),
            pltpu.VMEM((K, RPC, v), jnp.float32),
            pltpu.SemaphoreType.DMA((K,)),
            pltpu.SemaphoreType.DMA((K,)),
        ],
        compiler_params=pltpu.CompilerParams(
            dimension_semantics=("arbitrary",),
        ),
    )(logits)
